# baseline (speedup 1.0000x reference)
import functools

import jax
import jax.numpy as jnp
from jax import lax
from jax.experimental import pallas as pl
from jax.experimental.pallas import tpu as pltpu
from jax.experimental.pallas import tpu_sc as plsc

_N = 10000
_E = 320000
_H = 128
_DOUT = 64

_NCORES = 2
_NSUB = 16
_NTILES = _NCORES * _NSUB
_EPT = _E // _NTILES
_EC = 40
_NECR = _EPT // _EC
_NEC = 256
_NP = _N + 16
_NBUF = 4
_BLKC = 8
_NBLK = _NEC // _BLKC
_RC = 40
_NRC = _N // _RC
_RCPS = (_NRC + _NSUB - 1) // _NSUB

_BLK = 1000
_NB = _N // _BLK


def _segsum_body(h_hbm, src_hbm, dst_hbm, out_hbm, src_i, dst_i, rows_v,
                 acc, gsem, ssem, isem):
    c = lax.axis_index("c")
    s = lax.axis_index("s")
    wid = c * _NSUB + s

    pltpu.async_copy(src_hbm.at[wid, pl.ds(0, _BLKC)], src_i.at[0], isem)
    pltpu.async_copy(dst_hbm.at[wid, pl.ds(0, _BLKC)], dst_i.at[0], isem)

    zeros16 = jnp.zeros((16,), jnp.float32)

    def _zrow(i, carry):
        for j in range(_H // 16):
            rows_v[0, i, pl.ds(j * 16, 16)] = zeros16
        return carry

    lax.fori_loop(0, _RC, _zrow, 0)

    zd = []
    for j in range(_RCPS):
        chunk = j * _NSUB + s

        @pl.when(chunk < _NRC)
        def _(chunk=chunk):
            pltpu.async_copy(rows_v.at[0], acc.at[pl.ds(chunk * _RC, _RC)],
                             ssem)

        zd.append(chunk)
    for chunk in zd:
        @pl.when(chunk < _NRC)
        def _():
            pltpu.make_async_copy(h_hbm.at[pl.ds(0, _RC)], rows_v.at[0],
                                  ssem).wait()

    plsc.subcore_barrier()

    def _eblock(k, par):
        pltpu.make_async_copy(src_hbm.at[wid, pl.ds(0, _BLKC)],
                              src_i.at[par], isem).wait()
        pltpu.make_async_copy(dst_hbm.at[wid, pl.ds(0, _BLKC)],
                              dst_i.at[par], isem).wait()

        @pl.when(k + 1 < _NBLK)
        def _():
            nb = (k + 1) * _BLKC
            pltpu.async_copy(src_hbm.at[wid, pl.ds(nb, _BLKC)],
                             src_i.at[1 - par], isem)
            pltpu.async_copy(dst_hbm.at[wid, pl.ds(nb, _BLKC)],
                             dst_i.at[1 - par], isem)

        for half in range(_BLKC // _NBUF):
            gd = [pltpu.async_copy(h_hbm.at[src_i.at[par, half * _NBUF + b]],
                                   rows_v.at[b], gsem)
                  for b in range(_NBUF)]
            for d in gd:
                d.wait()
            sd = [pltpu.async_copy(rows_v.at[b],
                                   acc.at[dst_i.at[par, half * _NBUF + b]],
                                   ssem, add=True)
                  for b in range(_NBUF)]
            for d in sd:
                d.wait()

    def _epair(i, carry):
        _eblock(2 * i, 0)
        _eblock(2 * i + 1, 1)
        return carry

    lax.fori_loop(0, _NBLK // 2, _epair, 0)
    plsc.subcore_barrier()

    ngrp_o = (_RCPS + _NBUF - 1) // _NBUF
    for g in range(ngrp_o):
        js = [j for j in range(g * _NBUF, min((g + 1) * _NBUF, _RCPS))]
        for j in js:
            chunk = j * _NSUB + s

            @pl.when(chunk < _NRC)
            def _(chunk=chunk, b=j % _NBUF):
                r0 = chunk * _RC
                pltpu.sync_copy(acc.at[pl.ds(r0, _RC)], rows_v.at[b])
                pltpu.async_copy(rows_v.at[b],
                                 out_hbm.at[pl.ds(c * _N + r0, _RC)], gsem)

        for j in js:
            chunk = j * _NSUB + s

            @pl.when(chunk < _NRC)
            def _(b=j % _NBUF):
                pltpu.make_async_copy(h_hbm.at[pl.ds(0, _RC)], rows_v.at[b],
                                      gsem).wait()


@functools.cache
def _make_segsum():
    return functools.partial(
        pl.kernel,
        mesh=plsc.VectorSubcoreMesh(core_axis_name="c", subcore_axis_name="s"),
        out_type=jax.ShapeDtypeStruct((2 * _N, _H), jnp.float32),
        scratch_types=[
            pltpu.VMEM((2, _BLKC, _EC), jnp.int32),
            pltpu.VMEM((2, _BLKC, _EC), jnp.int32),
            pltpu.VMEM((_NBUF, _EC, _H), jnp.float32),
            pltpu.VMEM_SHARED((_NP, _H), jnp.float32),
            pltpu.SemaphoreType.DMA,
            pltpu.SemaphoreType.DMA,
            pltpu.SemaphoreType.DMA,
        ],
    )(_segsum_body)


def _round_a_body(h_ref, p_ref, wa_ref, ba_ref, wr_ref, br_ref,
                  t_ref, id_ref, st_ref):
    j = pl.program_id(0)
    h = h_ref[...]
    agg = h + p_ref[0] + p_ref[1]
    t = jnp.dot(agg, wa_ref[...], preferred_element_type=jnp.float32) + ba_ref[...]
    t_ref[...] = t
    id_ref[...] = jnp.dot(h, wr_ref[...], preferred_element_type=jnp.float32) + br_ref[...]
    blk = jnp.concatenate(
        [jnp.sum(t, axis=0, keepdims=True),
         jnp.sum(t * t, axis=0, keepdims=True)], axis=0)

    @pl.when(j == 0)
    def _():
        st_ref[...] = blk

    @pl.when(j > 0)
    def _():
        st_ref[...] += blk


def _round_a(h, parts, wa, ba, wr, br):
    return pl.pallas_call(
        _round_a_body,
        grid=(_NB,),
        in_specs=[
            pl.BlockSpec((_BLK, _H), lambda j: (j, 0)),
            pl.BlockSpec((2, _BLK, _H), lambda j: (0, j, 0)),
            pl.BlockSpec((_H, _H), lambda j: (0, 0)),
            pl.BlockSpec((1, _H), lambda j: (0, 0)),
            pl.BlockSpec((_H, _H), lambda j: (0, 0)),
            pl.BlockSpec((1, _H), lambda j: (0, 0)),
        ],
        out_specs=[
            pl.BlockSpec((_BLK, _H), lambda j: (j, 0)),
            pl.BlockSpec((_BLK, _H), lambda j: (j, 0)),
            pl.BlockSpec((2, _H), lambda j: (0, 0)),
        ],
        out_shape=[
            jax.ShapeDtypeStruct((_N, _H), jnp.float32),
            jax.ShapeDtypeStruct((_N, _H), jnp.float32),
            jax.ShapeDtypeStruct((2, _H), jnp.float32),
        ],
    )(h, parts, wa, ba.reshape(1, _H), wr, br.reshape(1, _H))


def _round_b_body(t_ref, id_ref, st_ref, g_ref, be_ref, wb_ref, bb_ref,
                  wc_ref, bc_ref, o_ref):
    st = st_ref[...]
    m = st[0:1] * (1.0 / _N)
    v = st[1:2] * (1.0 / _N) - m * m
    inv = lax.rsqrt(v + 1e-5)
    u = (t_ref[...] - m) * (inv * g_ref[...]) + be_ref[...]
    u = jnp.maximum(u, 0.0)
    u = jnp.maximum(
        jnp.dot(u, wb_ref[...], preferred_element_type=jnp.float32) + bb_ref[...], 0.0)
    hh = jnp.dot(u, wc_ref[...], preferred_element_type=jnp.float32) + bc_ref[...]
    o_ref[...] = jnp.maximum(hh + id_ref[...], 0.0)


def _round_b(t, ident, stats, g, be, wb, bb, wc, bc):
    return pl.pallas_call(
        _round_b_body,
        grid=(_NB,),
        in_specs=[
            pl.BlockSpec((_BLK, _H), lambda j: (j, 0)),
            pl.BlockSpec((_BLK, _H), lambda j: (j, 0)),
            pl.BlockSpec((2, _H), lambda j: (0, 0)),
            pl.BlockSpec((1, _H), lambda j: (0, 0)),
            pl.BlockSpec((1, _H), lambda j: (0, 0)),
            pl.BlockSpec((_H, _H), lambda j: (0, 0)),
            pl.BlockSpec((1, _H), lambda j: (0, 0)),
            pl.BlockSpec((_H, _H), lambda j: (0, 0)),
            pl.BlockSpec((1, _H), lambda j: (0, 0)),
        ],
        out_specs=pl.BlockSpec((_BLK, _H), lambda j: (j, 0)),
        out_shape=jax.ShapeDtypeStruct((_N, _H), jnp.float32),
    )(t, ident, stats, g.reshape(1, _H), be.reshape(1, _H),
      wb, bb.reshape(1, _H), wc, bc.reshape(1, _H))


def _final_body(h_ref, p_ref, w4_ref, b4_ref, o_ref):
    agg = h_ref[...] + p_ref[0] + p_ref[1]
    z = jnp.dot(agg, w4_ref[...], preferred_element_type=jnp.float32) + b4_ref[...]
    mx = jnp.max(z, axis=1, keepdims=True)
    e = jnp.exp(z - mx)
    o_ref[...] = z - mx - jnp.log(jnp.sum(e, axis=1, keepdims=True))


def _final(h, parts, w4, b4):
    return pl.pallas_call(
        _final_body,
        grid=(_NB,),
        in_specs=[
            pl.BlockSpec((_BLK, _H), lambda j: (j, 0)),
            pl.BlockSpec((2, _BLK, _H), lambda j: (0, j, 0)),
            pl.BlockSpec((_H, _DOUT), lambda j: (0, 0)),
            pl.BlockSpec((1, _DOUT), lambda j: (0, 0)),
        ],
        out_specs=pl.BlockSpec((_BLK, _DOUT), lambda j: (j, 0)),
        out_shape=jax.ShapeDtypeStruct((_N, _DOUT), jnp.float32),
    )(h, parts, w4, b4.reshape(1, _DOUT))


def kernel(x, edge_index,
           W1a, b1a, g1, be1, W1b, b1b, W1c, b1c,
           W2a, b2a, g2, be2, W2b, b2b, W2c, b2c,
           W3a, b3a, g3, be3, W3b, b3b, W3c, b3c,
           W4, b4, Wr1, br1, Wr2, br2, Wr3, br3):
    npad = _NEC - _NECR
    src = jnp.concatenate(
        [edge_index[0].reshape(_NTILES, _NECR, _EC),
         jnp.zeros((_NTILES, npad, _EC), jnp.int32)], axis=1)
    dst = jnp.concatenate(
        [edge_index[1].reshape(_NTILES, _NECR, _EC),
         jnp.full((_NTILES, npad, _EC), _N, jnp.int32)], axis=1)

    segsum = _make_segsum()

    def agg_parts(h):
        return segsum(h, src, dst).reshape(2, _N, _H)

    h = x
    for (wa, ba, g, be, wb, bb, wc, bc, wr, br) in (
            (W1a, b1a, g1, be1, W1b, b1b, W1c, b1c, Wr1, br1),
            (W2a, b2a, g2, be2, W2b, b2b, W2c, b2c, Wr2, br2),
            (W3a, b3a, g3, be3, W3b, b3b, W3c, b3c, Wr3, br3)):
        parts = agg_parts(h)
        t, ident, stats = _round_a(h, parts, wa, ba, wr, br)
        h = _round_b(t, ident, stats, g, be, wb, bb, wc, bc)
    return _final(h, agg_parts(h), W4, b4)


# speedup vs baseline: 1.1512x; 1.1512x over previous
import functools

import jax
import jax.numpy as jnp
from jax import lax
from jax.experimental import pallas as pl
from jax.experimental.pallas import tpu as pltpu
from jax.experimental.pallas import tpu_sc as plsc

_N = 10000
_E = 320000
_H = 128
_DOUT = 64

_NCORES = 2
_NSUB = 16
_NTILES = _NCORES * _NSUB
_EPT = _E // _NTILES
_EC = 80
_NECR = _EPT // _EC
_NEC = 128
_SBC = 16
_NSB = _NEC // _SBC
_NP = _N + 16
_NBUF = 3
_RC = 80
_NRC = _N // _RC
_RCPS = (_NRC + _NSUB - 1) // _NSUB

_BLK = 1000
_NB = _N // _BLK


def _segsum_body(h_hbm, src_hbm, dst_hbm, out_hbm, src_i, dst_i, rows_v,
                 acc, gs0, gs1, gs2, ss0, ss1, ss2, isem):
    c = lax.axis_index("c")
    s = lax.axis_index("s")
    wid = c * _NSUB + s
    gsem = (gs0, gs1, gs2)
    ssem = (ss0, ss1, ss2)

    def fire_idx(sb, par):
        pltpu.async_copy(src_hbm.at[wid, pl.ds(sb * _SBC, _SBC)],
                         src_i.at[par], isem)
        pltpu.async_copy(dst_hbm.at[wid, pl.ds(sb * _SBC, _SBC)],
                         dst_i.at[par], isem)

    def drain_idx(par):
        pltpu.make_async_copy(src_hbm.at[wid, pl.ds(0, _SBC)],
                              src_i.at[par], isem).wait()
        pltpu.make_async_copy(dst_hbm.at[wid, pl.ds(0, _SBC)],
                              dst_i.at[par], isem).wait()

    def drain_rowcopy(sem, b):
        pltpu.make_async_copy(h_hbm.at[pl.ds(0, _EC)], rows_v.at[b],
                              sem).wait()

    fire_idx(0, 0)

    zeros16 = jnp.zeros((16,), jnp.float32)

    def _zrow(i, carry):
        for j in range(_H // 16):
            rows_v[0, i, pl.ds(j * 16, 16)] = zeros16
        return carry

    lax.fori_loop(0, _RC, _zrow, 0)

    for j in range(_RCPS):
        chunk = j * _NSUB + s

        @pl.when(chunk < _NRC)
        def _(chunk=chunk):
            pltpu.async_copy(rows_v.at[0], acc.at[pl.ds(chunk * _RC, _RC)],
                             ss0)

    drain_idx(0)
    fire_idx(1, 1)
    for j in range(_RCPS):
        chunk = j * _NSUB + s

        @pl.when(chunk < _NRC)
        def _():
            drain_rowcopy(ss0, 0)

    plsc.subcore_barrier()

    pltpu.async_copy(h_hbm.at[src_i.at[0, 0]], rows_v.at[0], gs0)
    pltpu.async_copy(h_hbm.at[src_i.at[0, 1]], rows_v.at[1], gs1)
    sc_drained = 0
    for j in range(_NEC):
        sb, t = divmod(j, _SBC)
        par = sb % 2
        b = j % _NBUF
        drain_rowcopy(gsem[b], b)
        pltpu.async_copy(rows_v.at[b], acc.at[dst_i.at[par, t]], ssem[b],
                         add=True)
        if t == 0 and 0 < j and sb + 1 < _NSB:
            while sc_drained < j:
                drain_rowcopy(ssem[sc_drained % _NBUF], sc_drained % _NBUF)
                sc_drained += 1
            fire_idx(sb + 1, (sb + 1) % 2)
        nx = j + 2
        if nx < _NEC:
            nsb, nt = divmod(nx, _SBC)
            if nt == 0:
                drain_idx(nsb % 2)
            while sc_drained < j:
                drain_rowcopy(ssem[sc_drained % _NBUF], sc_drained % _NBUF)
                sc_drained += 1
            pltpu.async_copy(h_hbm.at[src_i.at[nsb % 2, nt]],
                             rows_v.at[nx % _NBUF], gsem[nx % _NBUF])
    while sc_drained < _NEC:
        drain_rowcopy(ssem[sc_drained % _NBUF], sc_drained % _NBUF)
        sc_drained += 1
    plsc.subcore_barrier()

    ngrp_o = (_RCPS + _NBUF - 1) // _NBUF
    for g in range(ngrp_o):
        js = [j for j in range(g * _NBUF, min((g + 1) * _NBUF, _RCPS))]
        for j in js:
            chunk = j * _NSUB + s

            @pl.when(chunk < _NRC)
            def _(chunk=chunk, b=j % _NBUF):
                r0 = chunk * _RC
                pltpu.sync_copy(acc.at[pl.ds(r0, _RC)], rows_v.at[b])
                pltpu.async_copy(rows_v.at[b],
                                 out_hbm.at[pl.ds(c * _N + r0, _RC)],
                                 gsem[b])

        for j in js:
            chunk = j * _NSUB + s

            @pl.when(chunk < _NRC)
            def _(b=j % _NBUF):
                drain_rowcopy(gsem[b], b)


@functools.cache
def _make_segsum():
    return functools.partial(
        pl.kernel,
        mesh=plsc.VectorSubcoreMesh(core_axis_name="c", subcore_axis_name="s"),
        out_type=jax.ShapeDtypeStruct((2 * _N, _H), jnp.float32),
        scratch_types=[
            pltpu.VMEM((2, _SBC, _EC), jnp.int32),
            pltpu.VMEM((2, _SBC, _EC), jnp.int32),
            pltpu.VMEM((_NBUF, _EC, _H), jnp.float32),
            pltpu.VMEM_SHARED((_NP, _H), jnp.float32),
            pltpu.SemaphoreType.DMA,
            pltpu.SemaphoreType.DMA,
            pltpu.SemaphoreType.DMA,
            pltpu.SemaphoreType.DMA,
            pltpu.SemaphoreType.DMA,
            pltpu.SemaphoreType.DMA,
            pltpu.SemaphoreType.DMA,
        ],
    )(_segsum_body)


def _round_a_body(h_ref, p_ref, wa_ref, ba_ref, wr_ref, br_ref,
                  t_ref, id_ref, st_ref):
    j = pl.program_id(0)
    h = h_ref[...]
    agg = h + p_ref[0] + p_ref[1]
    t = jnp.dot(agg, wa_ref[...], preferred_element_type=jnp.float32) + ba_ref[...]
    t_ref[...] = t
    id_ref[...] = jnp.dot(h, wr_ref[...], preferred_element_type=jnp.float32) + br_ref[...]
    blk = jnp.concatenate(
        [jnp.sum(t, axis=0, keepdims=True),
         jnp.sum(t * t, axis=0, keepdims=True)], axis=0)

    @pl.when(j == 0)
    def _():
        st_ref[...] = blk

    @pl.when(j > 0)
    def _():
        st_ref[...] += blk


def _round_a(h, parts, wa, ba, wr, br):
    return pl.pallas_call(
        _round_a_body,
        grid=(_NB,),
        in_specs=[
            pl.BlockSpec((_BLK, _H), lambda j: (j, 0)),
            pl.BlockSpec((2, _BLK, _H), lambda j: (0, j, 0)),
            pl.BlockSpec((_H, _H), lambda j: (0, 0)),
            pl.BlockSpec((1, _H), lambda j: (0, 0)),
            pl.BlockSpec((_H, _H), lambda j: (0, 0)),
            pl.BlockSpec((1, _H), lambda j: (0, 0)),
        ],
        out_specs=[
            pl.BlockSpec((_BLK, _H), lambda j: (j, 0)),
            pl.BlockSpec((_BLK, _H), lambda j: (j, 0)),
            pl.BlockSpec((2, _H), lambda j: (0, 0)),
        ],
        out_shape=[
            jax.ShapeDtypeStruct((_N, _H), jnp.float32),
            jax.ShapeDtypeStruct((_N, _H), jnp.float32),
            jax.ShapeDtypeStruct((2, _H), jnp.float32),
        ],
    )(h, parts, wa, ba.reshape(1, _H), wr, br.reshape(1, _H))


def _round_b_body(t_ref, id_ref, st_ref, g_ref, be_ref, wb_ref, bb_ref,
                  wc_ref, bc_ref, o_ref):
    st = st_ref[...]
    m = st[0:1] * (1.0 / _N)
    v = st[1:2] * (1.0 / _N) - m * m
    inv = lax.rsqrt(v + 1e-5)
    u = (t_ref[...] - m) * (inv * g_ref[...]) + be_ref[...]
    u = jnp.maximum(u, 0.0)
    u = jnp.maximum(
        jnp.dot(u, wb_ref[...], preferred_element_type=jnp.float32) + bb_ref[...], 0.0)
    hh = jnp.dot(u, wc_ref[...], preferred_element_type=jnp.float32) + bc_ref[...]
    o_ref[...] = jnp.maximum(hh + id_ref[...], 0.0)


def _round_b(t, ident, stats, g, be, wb, bb, wc, bc):
    return pl.pallas_call(
        _round_b_body,
        grid=(_NB,),
        in_specs=[
            pl.BlockSpec((_BLK, _H), lambda j: (j, 0)),
            pl.BlockSpec((_BLK, _H), lambda j: (j, 0)),
            pl.BlockSpec((2, _H), lambda j: (0, 0)),
            pl.BlockSpec((1, _H), lambda j: (0, 0)),
            pl.BlockSpec((1, _H), lambda j: (0, 0)),
            pl.BlockSpec((_H, _H), lambda j: (0, 0)),
            pl.BlockSpec((1, _H), lambda j: (0, 0)),
            pl.BlockSpec((_H, _H), lambda j: (0, 0)),
            pl.BlockSpec((1, _H), lambda j: (0, 0)),
        ],
        out_specs=pl.BlockSpec((_BLK, _H), lambda j: (j, 0)),
        out_shape=jax.ShapeDtypeStruct((_N, _H), jnp.float32),
    )(t, ident, stats, g.reshape(1, _H), be.reshape(1, _H),
      wb, bb.reshape(1, _H), wc, bc.reshape(1, _H))


def _final_body(h_ref, p_ref, w4_ref, b4_ref, o_ref):
    agg = h_ref[...] + p_ref[0] + p_ref[1]
    z = jnp.dot(agg, w4_ref[...], preferred_element_type=jnp.float32) + b4_ref[...]
    mx = jnp.max(z, axis=1, keepdims=True)
    e = jnp.exp(z - mx)
    o_ref[...] = z - mx - jnp.log(jnp.sum(e, axis=1, keepdims=True))


def _final(h, parts, w4, b4):
    return pl.pallas_call(
        _final_body,
        grid=(_NB,),
        in_specs=[
            pl.BlockSpec((_BLK, _H), lambda j: (j, 0)),
            pl.BlockSpec((2, _BLK, _H), lambda j: (0, j, 0)),
            pl.BlockSpec((_H, _DOUT), lambda j: (0, 0)),
            pl.BlockSpec((1, _DOUT), lambda j: (0, 0)),
        ],
        out_specs=pl.BlockSpec((_BLK, _DOUT), lambda j: (j, 0)),
        out_shape=jax.ShapeDtypeStruct((_N, _DOUT), jnp.float32),
    )(h, parts, w4, b4.reshape(1, _DOUT))


def kernel(x, edge_index,
           W1a, b1a, g1, be1, W1b, b1b, W1c, b1c,
           W2a, b2a, g2, be2, W2b, b2b, W2c, b2c,
           W3a, b3a, g3, be3, W3b, b3b, W3c, b3c,
           W4, b4, Wr1, br1, Wr2, br2, Wr3, br3):
    npad = _NEC - _NECR
    src = jnp.concatenate(
        [edge_index[0].reshape(_NTILES, _NECR, _EC),
         jnp.zeros((_NTILES, npad, _EC), jnp.int32)], axis=1)
    dst = jnp.concatenate(
        [edge_index[1].reshape(_NTILES, _NECR, _EC),
         jnp.full((_NTILES, npad, _EC), _N, jnp.int32)], axis=1)

    segsum = _make_segsum()

    def agg_parts(h):
        return segsum(h, src, dst).reshape(2, _N, _H)

    h = x
    for (wa, ba, g, be, wb, bb, wc, bc, wr, br) in (
            (W1a, b1a, g1, be1, W1b, b1b, W1c, b1c, Wr1, br1),
            (W2a, b2a, g2, be2, W2b, b2b, W2c, b2c, Wr2, br2),
            (W3a, b3a, g3, be3, W3b, b3b, W3c, b3c, Wr3, br3)):
        parts = agg_parts(h)
        t, ident, stats = _round_a(h, parts, wa, ba, wr, br)
        h = _round_b(t, ident, stats, g, be, wb, bb, wc, bc)
    return _final(h, agg_parts(h), W4, b4)
